# baseline (speedup 1.0000x reference)
import functools

import jax
import jax.numpy as jnp
from jax import lax
from jax.experimental import pallas as pl
from jax.experimental.pallas import tpu as pltpu
from jax.experimental.pallas import tpu_sc as plsc

VOCAB = 100000
EMBED_DIM = 64
BATCH = 4096
HIST_LEN = 200

N = BATCH * HIST_LEN
NC, NS = 2, 16
NW = NC * NS
B_PER_W = BATCH // NW
F_PER_W = B_PER_W * HIST_LEN
BB = 2
CHUNK = BB * HIST_LEN
N_CHUNKS = B_PER_W // BB
ROW_SPLITS = [(0, 128), (128, 72)]


def _make_embed():
    mesh = plsc.VectorSubcoreMesh(core_axis_name="c", subcore_axis_name="s")

    @functools.partial(
        pl.kernel,
        mesh=mesh,
        compiler_params=pltpu.CompilerParams(use_tc_tiling_on_sc=False),
        out_type=jax.ShapeDtypeStruct(
            (BATCH // 2, HIST_LEN, 2 * EMBED_DIM), jnp.float32),
        scratch_types=[
            pltpu.VMEM((BB, HIST_LEN), jnp.int32),
            pltpu.VMEM((BB, HIST_LEN), jnp.int32),
            pltpu.VMEM((BB, HIST_LEN, EMBED_DIM), jnp.float32),
            pltpu.VMEM((BB, HIST_LEN, EMBED_DIM), jnp.float32),
            pltpu.SemaphoreType.DMA,
            pltpu.SemaphoreType.DMA,
            pltpu.SemaphoreType.DMA,
            pltpu.SemaphoreType.DMA,
        ],
    )
    def embed(x_hbm, table_hbm, out_hbm, idx0, idx1, rows0, rows1,
              g0, g1, o0, o1):
        wid = lax.axis_index("s") * NC + lax.axis_index("c")
        batch_base = wid * B_PER_W
        col0 = (wid // (NW // 2)) * EMBED_DIM
        local_batch = (wid % (NW // 2)) * B_PER_W
        idx_v = (idx0, idx1)
        rows_v = (rows0, rows1)
        gsem = (g0, g1)
        osem = (o0, o1)

        def load_idx(c, b):
            bi = batch_base + c * BB
            pltpu.sync_copy(x_hbm.at[pl.ds(bi, BB)], idx_v[b])

        def start_gather(b):
            for r in range(BB):
                for off, ln in ROW_SPLITS:
                    pltpu.async_copy(
                        table_hbm.at[idx_v[b].at[r, pl.ds(off, ln)]],
                        rows_v[b].at[r, pl.ds(off, ln)],
                        gsem[b],
                    )

        def wait_gather(b):
            for r in range(BB):
                for off, ln in ROW_SPLITS:
                    pltpu.make_async_copy(
                        table_hbm.at[idx_v[b].at[r, pl.ds(off, ln)]],
                        rows_v[b].at[r, pl.ds(off, ln)],
                        gsem[b],
                    ).wait()

        def start_write(c, b):
            bb = local_batch + c * BB
            pltpu.async_copy(
                rows_v[b],
                out_hbm.at[pl.ds(bb, BB), :, pl.ds(col0, EMBED_DIM)],
                osem[b],
            )

        def wait_write(b):
            pltpu.make_async_copy(
                rows_v[b],
                out_hbm.at[pl.ds(0, BB), :, pl.ds(col0, EMBED_DIM)],
                osem[b],
            ).wait()

        def step(c, b, do_wait_write, do_next_gather, do_prefetch):
            if do_next_gather:
                if do_wait_write:
                    wait_write(1 - b)
                start_gather(1 - b)
            wait_gather(b)
            start_write(c, b)
            if do_prefetch:
                load_idx(c + 2, b)

        load_idx(0, 0)
        start_gather(0)
        load_idx(1, 1)

        step(0, 0, False, True, True)
        step(1, 1, True, True, True)

        def interior(t, carry):
            step(2 * t, 0, True, True, True)
            step(2 * t + 1, 1, True, True, True)
            return carry

        lax.fori_loop(1, N_CHUNKS // 2 - 1, interior, 0)

        step(N_CHUNKS - 2, 0, True, True, False)
        step(N_CHUNKS - 1, 1, True, False, False)
        wait_write(0)
        wait_write(1)

    return embed


_embed = _make_embed()

_RL_BB = 8
_RL_BLOCKS = BATCH // 2 // _RL_BB


def _relayout_body(i_ref, o_ref):
    h = pl.program_id(1)

    @pl.when(h == 0)
    def _():
        for k in range(_RL_BB):
            o_ref[k] = i_ref[k, :, :EMBED_DIM]

    @pl.when(h == 1)
    def _():
        for k in range(_RL_BB):
            o_ref[k] = i_ref[k, :, EMBED_DIM:]


_relayout = pl.pallas_call(
    _relayout_body,
    grid=(_RL_BLOCKS, 2),
    in_specs=[pl.BlockSpec((_RL_BB, HIST_LEN, 2 * EMBED_DIM),
                           lambda i, h: (i, 0, 0))],
    out_specs=pl.BlockSpec((_RL_BB, HIST_LEN, EMBED_DIM),
                           lambda i, h: (h * _RL_BLOCKS + i, 0, 0)),
    out_shape=jax.ShapeDtypeStruct((BATCH, HIST_LEN, EMBED_DIM), jnp.float32),
)


def kernel(x, table):
    halves = _embed(x.astype(jnp.int32), table)
    return _relayout(halves)


# speedup vs baseline: 1.3878x; 1.3878x over previous
import functools

import jax
import jax.numpy as jnp
from jax import lax
from jax.experimental import pallas as pl
from jax.experimental.pallas import tpu as pltpu
from jax.experimental.pallas import tpu_sc as plsc

VOCAB = 100000
EMBED_DIM = 64
BATCH = 4096
HIST_LEN = 200

NC, NS = 2, 16
NW = NC * NS
B_PER_W = BATCH // NW
BB = 2
CHUNK = BB * HIST_LEN
N_CHUNKS = B_PER_W // BB
ROW_SPLITS = [(0, 128), (128, 72)]


def _make_embed():
    mesh = plsc.VectorSubcoreMesh(core_axis_name="c", subcore_axis_name="s")

    @functools.partial(
        pl.kernel,
        mesh=mesh,
        compiler_params=pltpu.CompilerParams(use_tc_tiling_on_sc=False),
        out_type=jax.ShapeDtypeStruct(
            (BATCH, HIST_LEN, EMBED_DIM), jnp.float32),
        scratch_types=[
            pltpu.VMEM((BB, HIST_LEN), jnp.int32),
            pltpu.VMEM((BB, HIST_LEN), jnp.int32),
            pltpu.VMEM((BB, HIST_LEN, EMBED_DIM), jnp.float32),
            pltpu.VMEM((BB, HIST_LEN, EMBED_DIM), jnp.float32),
            pltpu.SemaphoreType.DMA,
            pltpu.SemaphoreType.DMA,
            pltpu.SemaphoreType.DMA,
            pltpu.SemaphoreType.DMA,
        ],
    )
    def embed(x_hbm, table_hbm, out_hbm, idx0, idx1, rows0, rows1,
              g0, g1, o0, o1):
        wid = lax.axis_index("s") * NC + lax.axis_index("c")
        batch_base = wid * B_PER_W
        idx_v = (idx0, idx1)
        rows_v = (rows0, rows1)
        gsem = (g0, g1)
        osem = (o0, o1)

        def load_idx(c, b):
            bi = batch_base + c * BB
            pltpu.sync_copy(x_hbm.at[pl.ds(bi, BB)], idx_v[b])

        def start_gather(b):
            for r in range(BB):
                for off, ln in ROW_SPLITS:
                    pltpu.async_copy(
                        table_hbm.at[idx_v[b].at[r, pl.ds(off, ln)]],
                        rows_v[b].at[r, pl.ds(off, ln)],
                        gsem[b],
                    )

        def wait_gather(b):
            for r in range(BB):
                for off, ln in ROW_SPLITS:
                    pltpu.make_async_copy(
                        table_hbm.at[idx_v[b].at[r, pl.ds(off, ln)]],
                        rows_v[b].at[r, pl.ds(off, ln)],
                        gsem[b],
                    ).wait()

        def start_write(c, b):
            bi = batch_base + c * BB
            pltpu.async_copy(rows_v[b], out_hbm.at[pl.ds(bi, BB)], osem[b])

        def wait_write(b):
            pltpu.make_async_copy(
                rows_v[b], out_hbm.at[pl.ds(0, BB)], osem[b]
            ).wait()

        def step(c, b, do_wait_write, do_next_gather, do_prefetch):
            if do_next_gather:
                if do_wait_write:
                    wait_write(1 - b)
                start_gather(1 - b)
            wait_gather(b)
            start_write(c, b)
            if do_prefetch:
                load_idx(c + 2, b)

        load_idx(0, 0)
        start_gather(0)
        load_idx(1, 1)

        step(0, 0, False, True, True)
        step(1, 1, True, True, True)

        def interior(t, carry):
            step(2 * t, 0, True, True, True)
            step(2 * t + 1, 1, True, True, True)
            return carry

        lax.fori_loop(1, N_CHUNKS // 2 - 1, interior, 0)

        step(N_CHUNKS - 2, 0, True, True, False)
        step(N_CHUNKS - 1, 1, True, False, False)
        wait_write(0)
        wait_write(1)

    return embed


_embed = _make_embed()


def kernel(x, table):
    return _embed(x.astype(jnp.int32), table)
